# baseline (speedup 1.0000x reference)
import functools

import jax
import jax.numpy as jnp
from jax import lax
from jax.experimental import pallas as pl
from jax.experimental.pallas import tpu as pltpu
from jax.experimental.pallas import tpu_sc as plsc

_BATCH = 16384
_EMB = 32
_NC = 2
_NS = 16
_NW = _NC * _NS
_BPW = _BATCH // _NW
_CHUNK = 128
_NCHUNK = _BPW // _CHUNK


def _gather_body(uid_hbm, iid_hbm, ut_hbm, it_hbm, ue_hbm, ie_hbm,
                 uidx_v, iidx_v, urows_v, irows_v, usem, isem):
    wid = lax.axis_index("s") * _NC + lax.axis_index("c")
    base = wid * _BPW
    for j in range(_NCHUNK):
        pltpu.sync_copy(uid_hbm.at[pl.ds(base + j * _CHUNK, _CHUNK)], uidx_v.at[j])
        pltpu.sync_copy(iid_hbm.at[pl.ds(base + j * _CHUNK, _CHUNK)], iidx_v.at[j])
    copies = []
    for j in range(_NCHUNK):
        dst = pl.ds(j * _CHUNK, _CHUNK)
        copies.append(pltpu.async_copy(ut_hbm.at[uidx_v.at[j]], urows_v.at[dst], usem))
        copies.append(pltpu.async_copy(it_hbm.at[iidx_v.at[j]], irows_v.at[dst], isem))
    for c in copies:
        c.wait()
    pltpu.sync_copy(urows_v, ue_hbm.at[pl.ds(base, _BPW)])
    pltpu.sync_copy(irows_v, ie_hbm.at[pl.ds(base, _BPW)])


@jax.jit
def _gather(uid, iid, user_table, item_table):
    mesh = plsc.VectorSubcoreMesh(core_axis_name="c", subcore_axis_name="s")
    k = pl.kernel(
        _gather_body,
        mesh=mesh,
        compiler_params=pltpu.CompilerParams(use_tc_tiling_on_sc=False),
        out_type=[
            jax.ShapeDtypeStruct((_BATCH, _EMB), jnp.float32),
            jax.ShapeDtypeStruct((_BATCH, _EMB), jnp.float32),
        ],
        scratch_types=[
            pltpu.VMEM((_NCHUNK, _CHUNK), jnp.int32),
            pltpu.VMEM((_NCHUNK, _CHUNK), jnp.int32),
            pltpu.VMEM((_BPW, _EMB), jnp.float32),
            pltpu.VMEM((_BPW, _EMB), jnp.float32),
            pltpu.SemaphoreType.DMA,
            pltpu.SemaphoreType.DMA,
        ],
    )
    return k(uid, iid, user_table, item_table)


_BLK = 2048


def _mlp_body(ue, ie, w1u, w1i, b1, w2, b2, w3, b3, wo, bo, out):
    h = jnp.dot(ue[...], w1u[...], preferred_element_type=jnp.float32)
    h = h + jnp.dot(ie[...], w1i[...], preferred_element_type=jnp.float32)
    h = jnp.maximum(h + b1[...], 0.0)
    h = jnp.maximum(jnp.dot(h, w2[...], preferred_element_type=jnp.float32) + b2[...], 0.0)
    h = jnp.maximum(jnp.dot(h, w3[...], preferred_element_type=jnp.float32) + b3[...], 0.0)
    z = jnp.sum(h * wo[...], axis=1) + bo[0, 0]
    out[...] = 1.0 / (1.0 + jnp.exp(-z))


@functools.partial(jax.jit, static_argnames=())
def _mlp(ue, ie, w1u, w1i, b1, w2, b2, w3, b3, wo, bo):
    grid = (_BATCH // _BLK,)
    full = lambda shape: pl.BlockSpec(shape, lambda i: (0,) * len(shape))
    return pl.pallas_call(
        _mlp_body,
        grid=grid,
        in_specs=[
            pl.BlockSpec((_BLK, _EMB), lambda i: (i, 0)),
            pl.BlockSpec((_BLK, _EMB), lambda i: (i, 0)),
            full(w1u.shape),
            full(w1i.shape),
            full(b1.shape),
            full(w2.shape),
            full(b2.shape),
            full(w3.shape),
            full(b3.shape),
            full(wo.shape),
            full(bo.shape),
        ],
        out_specs=pl.BlockSpec((_BLK,), lambda i: (i,)),
        out_shape=jax.ShapeDtypeStruct((_BATCH,), jnp.float32),
        compiler_params=pltpu.CompilerParams(
            dimension_semantics=("arbitrary",),
        ),
    )(ue, ie, w1u, w1i, b1, w2, b2, w3, b3, wo, bo)


def kernel(user_ids, item_ids, user_table, item_table, W1, b1, W2, b2, W3, b3, Wout, bout):
    uid = user_ids.astype(jnp.int32)
    iid = item_ids.astype(jnp.int32)
    ue, ie = _gather(uid, iid, user_table, item_table)
    return _mlp(
        ue, ie,
        W1[:_EMB], W1[_EMB:],
        b1.reshape(1, -1),
        W2, b2.reshape(1, -1),
        W3, b3.reshape(1, -1),
        Wout.T, bout.reshape(1, 1),
    )


# speedup vs baseline: 1.5023x; 1.5023x over previous
import jax
import jax.numpy as jnp
from jax import lax
from jax.experimental import pallas as pl
from jax.experimental.pallas import tpu as pltpu
from jax.experimental.pallas import tpu_sc as plsc

_BATCH = 16384
_EMB = 32
_NROWS = 1000000
_NC = 2
_NS = 16
_NW = _NC * _NS
_BPW = _BATCH // _NW
_CHUNK = 128
_NCHUNK = _BPW // _CHUNK
_LANES = 128

_TBLK = 2048
_TGRID = -(-_NROWS // (4 * _TBLK))
_Q = _TGRID * _TBLK


def _tr_body(x0, x1, x2, x3, out):
    out[...] = jnp.concatenate(
        [x0[...].T, x1[...].T, x2[...].T, x3[...].T], axis=1)


@jax.jit
def _pack(tab_t):
    tail = jnp.pad(tab_t[:, 3 * _Q:], ((0, 0), (0, 4 * _Q - _NROWS)))
    spec = lambda c: pl.BlockSpec((_EMB, _TBLK), lambda i, c=c: (0, i + _TGRID * c))
    return pl.pallas_call(
        _tr_body,
        grid=(_TGRID,),
        in_specs=[spec(0), spec(1), spec(2),
                  pl.BlockSpec((_EMB, _TBLK), lambda i: (0, i))],
        out_specs=pl.BlockSpec((_TBLK, _LANES), lambda i: (i, 0)),
        out_shape=jax.ShapeDtypeStruct((_Q, _LANES), jnp.float32),
        compiler_params=pltpu.CompilerParams(
            dimension_semantics=("arbitrary",),
        ),
    )(tab_t, tab_t, tab_t, tail)


def _gather_body(ids_hbm, tab_hbm, out_hbm, idx_v, gidx_v, buf, sem):
    wid = lax.axis_index("s") * _NC + lax.axis_index("c")
    base = wid * _BPW
    pltpu.sync_copy(ids_hbm.at[pl.ds(base, _BPW)], idx_v)
    for t in range(_BPW // 16):
        sl = pl.ds(t * 16, 16)
        r = idx_v[sl]
        c = ((r >= _Q).astype(jnp.int32) + (r >= 2 * _Q).astype(jnp.int32)
             + (r >= 3 * _Q).astype(jnp.int32))
        gidx_v[sl] = r - c * _Q
    handles = []
    for c in range(_NCHUNK):
        sl = pl.ds(c * _CHUNK, _CHUNK)
        handles.append(pltpu.async_copy(
            tab_hbm.at[gidx_v.at[sl]], buf.at[sl], sem))
    for h in handles:
        h.wait()
    pltpu.sync_copy(buf, out_hbm.at[pl.ds(base, _BPW)])


@jax.jit
def _gather(ids, tab2):
    mesh = plsc.VectorSubcoreMesh(core_axis_name="c", subcore_axis_name="s")
    k = pl.kernel(
        _gather_body,
        mesh=mesh,
        compiler_params=pltpu.CompilerParams(needs_layout_passes=False),
        out_type=jax.ShapeDtypeStruct((_BATCH, _LANES), jnp.float32),
        scratch_types=[
            pltpu.VMEM((_BPW,), jnp.int32),
            pltpu.VMEM((_BPW,), jnp.int32),
            pltpu.VMEM((_BPW, _LANES), jnp.float32),
            pltpu.SemaphoreType.DMA,
        ],
    )
    return k(ids, tab2)


_BLK = 2048


def _quarter(ids):
    return ((ids >= _Q).astype(jnp.int32) + (ids >= 2 * _Q).astype(jnp.int32)
            + (ids >= 3 * _Q).astype(jnp.int32))


def _select4(c, rows):
    return jnp.where(
        c < 2,
        jnp.where(c == 0, rows[:, 0 * _EMB:1 * _EMB], rows[:, 1 * _EMB:2 * _EMB]),
        jnp.where(c == 2, rows[:, 2 * _EMB:3 * _EMB], rows[:, 3 * _EMB:4 * _EMB]),
    )


def _mlp_body(uid_r, iid_r, ue, ie, w1u, w1i, b1, w2, b2, w3, b3, wo, bo, out):
    xu = _select4(_quarter(uid_r[...]), ue[...])
    xi = _select4(_quarter(iid_r[...]), ie[...])
    h = jnp.dot(xu, w1u[...], preferred_element_type=jnp.float32)
    h = h + jnp.dot(xi, w1i[...], preferred_element_type=jnp.float32)
    h = jnp.maximum(h + b1[...], 0.0)
    h = jnp.maximum(jnp.dot(h, w2[...], preferred_element_type=jnp.float32) + b2[...], 0.0)
    h = jnp.maximum(jnp.dot(h, w3[...], preferred_element_type=jnp.float32) + b3[...], 0.0)
    z = jnp.sum(h * wo[...], axis=1) + bo[0, 0]
    out[...] = 1.0 / (1.0 + jnp.exp(-z))


@jax.jit
def _mlp(uid2, iid2, ue, ie, w1u, w1i, b1, w2, b2, w3, b3, wo, bo):
    grid = (_BATCH // _BLK,)
    full = lambda shape: pl.BlockSpec(shape, lambda i: (0,) * len(shape))
    return pl.pallas_call(
        _mlp_body,
        grid=grid,
        in_specs=[
            pl.BlockSpec((_BLK, 1), lambda i: (i, 0)),
            pl.BlockSpec((_BLK, 1), lambda i: (i, 0)),
            pl.BlockSpec((_BLK, _LANES), lambda i: (i, 0)),
            pl.BlockSpec((_BLK, _LANES), lambda i: (i, 0)),
            full(w1u.shape),
            full(w1i.shape),
            full(b1.shape),
            full(w2.shape),
            full(b2.shape),
            full(w3.shape),
            full(b3.shape),
            full(wo.shape),
            full(bo.shape),
        ],
        out_specs=pl.BlockSpec((_BLK,), lambda i: (i,)),
        out_shape=jax.ShapeDtypeStruct((_BATCH,), jnp.float32),
        compiler_params=pltpu.CompilerParams(
            dimension_semantics=("arbitrary",),
        ),
    )(uid2, iid2, ue, ie, w1u, w1i, b1, w2, b2, w3, b3, wo, bo)


def kernel(user_ids, item_ids, user_table, item_table, W1, b1, W2, b2, W3, b3, Wout, bout):
    uid = user_ids.astype(jnp.int32)
    iid = item_ids.astype(jnp.int32)
    ut2 = _pack(user_table.T)
    it2 = _pack(item_table.T)
    ue = _gather(uid, ut2)
    ie = _gather(iid, it2)
    return _mlp(
        uid.reshape(-1, 1), iid.reshape(-1, 1), ue, ie,
        W1[:_EMB], W1[_EMB:],
        b1.reshape(1, -1),
        W2, b2.reshape(1, -1),
        W3, b3.reshape(1, -1),
        Wout.T, bout.reshape(1, 1),
    )


# speedup vs baseline: 2.3653x; 1.5744x over previous
import jax
import jax.numpy as jnp
from jax import lax
from jax.experimental import pallas as pl
from jax.experimental.pallas import tpu as pltpu
from jax.experimental.pallas import tpu_sc as plsc

_BATCH = 16384
_EMB = 32
_NROWS = 1000000
_NC = 2
_NS = 16
_NW = _NC * _NS
_BPW = _BATCH // _NW
_CHUNK = 128
_NCHUNK = _BPW // _CHUNK
_LANES = 128

_TBLK = 2048
_TGRID = -(-_NROWS // (4 * _TBLK))
_Q = _TGRID * _TBLK


def _tr_body(x0, x1, x2, x3, eye, out):
    xcat = jnp.concatenate([x0[...], x1[...], x2[...], x3[...]], axis=0)
    out[...] = jax.lax.dot_general(
        xcat, eye[...], (((0,), (0,)), ((), ())),
        preferred_element_type=jnp.float32)


@jax.jit
def _pack(tab_t):
    tail = jnp.pad(tab_t[:, 3 * _Q:], ((0, 0), (0, 4 * _Q - _NROWS)))
    spec = lambda c: pl.BlockSpec((_EMB, _TBLK), lambda i, c=c: (0, i + _TGRID * c))
    return pl.pallas_call(
        _tr_body,
        grid=(_TGRID,),
        in_specs=[spec(0), spec(1), spec(2),
                  pl.BlockSpec((_EMB, _TBLK), lambda i: (0, i)),
                  pl.BlockSpec((_LANES, _LANES), lambda i: (0, 0))],
        out_specs=pl.BlockSpec((_TBLK, _LANES), lambda i: (i, 0)),
        out_shape=jax.ShapeDtypeStruct((_Q, _LANES), jnp.float32),
        compiler_params=pltpu.CompilerParams(
            dimension_semantics=("arbitrary",),
        ),
    )(tab_t, tab_t, tab_t, tail, jnp.eye(_LANES, dtype=jnp.float32))


def _gather_body(ids_hbm, tab_hbm, out_hbm, idx_v, gidx_v, buf, sem):
    wid = lax.axis_index("s") * _NC + lax.axis_index("c")
    base = wid * _BPW
    pltpu.sync_copy(ids_hbm.at[pl.ds(base, _BPW)], idx_v)
    for t in range(_BPW // 16):
        sl = pl.ds(t * 16, 16)
        r = idx_v[sl]
        c = ((r >= _Q).astype(jnp.int32) + (r >= 2 * _Q).astype(jnp.int32)
             + (r >= 3 * _Q).astype(jnp.int32))
        gidx_v[sl] = r - c * _Q
    handles = []
    for c in range(_NCHUNK):
        sl = pl.ds(c * _CHUNK, _CHUNK)
        handles.append(pltpu.async_copy(
            tab_hbm.at[gidx_v.at[sl]], buf.at[sl], sem))
    for h in handles:
        h.wait()
    pltpu.sync_copy(buf, out_hbm.at[pl.ds(base, _BPW)])


@jax.jit
def _gather(ids, tab2):
    mesh = plsc.VectorSubcoreMesh(core_axis_name="c", subcore_axis_name="s")
    k = pl.kernel(
        _gather_body,
        mesh=mesh,
        compiler_params=pltpu.CompilerParams(needs_layout_passes=False),
        out_type=jax.ShapeDtypeStruct((_BATCH, _LANES), jnp.float32),
        scratch_types=[
            pltpu.VMEM((_BPW,), jnp.int32),
            pltpu.VMEM((_BPW,), jnp.int32),
            pltpu.VMEM((_BPW, _LANES), jnp.float32),
            pltpu.SemaphoreType.DMA,
        ],
    )
    return k(ids, tab2)


_BLK = 2048


def _quarter(ids):
    return ((ids >= _Q).astype(jnp.int32) + (ids >= 2 * _Q).astype(jnp.int32)
            + (ids >= 3 * _Q).astype(jnp.int32))


def _select4(c, rows):
    return jnp.where(
        c < 2,
        jnp.where(c == 0, rows[:, 0 * _EMB:1 * _EMB], rows[:, 1 * _EMB:2 * _EMB]),
        jnp.where(c == 2, rows[:, 2 * _EMB:3 * _EMB], rows[:, 3 * _EMB:4 * _EMB]),
    )


def _mlp_body(uid_r, iid_r, ue, ie, w1u, w1i, b1, w2, b2, w3, b3, wo, bo, out):
    xu = _select4(_quarter(uid_r[...]), ue[...])
    xi = _select4(_quarter(iid_r[...]), ie[...])
    h = jnp.dot(xu, w1u[...], preferred_element_type=jnp.float32)
    h = h + jnp.dot(xi, w1i[...], preferred_element_type=jnp.float32)
    h = jnp.maximum(h + b1[...], 0.0)
    h = jnp.maximum(jnp.dot(h, w2[...], preferred_element_type=jnp.float32) + b2[...], 0.0)
    h = jnp.maximum(jnp.dot(h, w3[...], preferred_element_type=jnp.float32) + b3[...], 0.0)
    z = jnp.sum(h * wo[...], axis=1) + bo[0, 0]
    out[...] = 1.0 / (1.0 + jnp.exp(-z))


@jax.jit
def _mlp(uid2, iid2, ue, ie, w1u, w1i, b1, w2, b2, w3, b3, wo, bo):
    grid = (_BATCH // _BLK,)
    full = lambda shape: pl.BlockSpec(shape, lambda i: (0,) * len(shape))
    return pl.pallas_call(
        _mlp_body,
        grid=grid,
        in_specs=[
            pl.BlockSpec((_BLK, 1), lambda i: (i, 0)),
            pl.BlockSpec((_BLK, 1), lambda i: (i, 0)),
            pl.BlockSpec((_BLK, _LANES), lambda i: (i, 0)),
            pl.BlockSpec((_BLK, _LANES), lambda i: (i, 0)),
            full(w1u.shape),
            full(w1i.shape),
            full(b1.shape),
            full(w2.shape),
            full(b2.shape),
            full(w3.shape),
            full(b3.shape),
            full(wo.shape),
            full(bo.shape),
        ],
        out_specs=pl.BlockSpec((_BLK,), lambda i: (i,)),
        out_shape=jax.ShapeDtypeStruct((_BATCH,), jnp.float32),
        compiler_params=pltpu.CompilerParams(
            dimension_semantics=("arbitrary",),
        ),
    )(uid2, iid2, ue, ie, w1u, w1i, b1, w2, b2, w3, b3, wo, bo)


def kernel(user_ids, item_ids, user_table, item_table, W1, b1, W2, b2, W3, b3, Wout, bout):
    uid = user_ids.astype(jnp.int32)
    iid = item_ids.astype(jnp.int32)
    ut2 = _pack(user_table.T)
    it2 = _pack(item_table.T)
    ue = _gather(uid, ut2)
    ie = _gather(iid, it2)
    return _mlp(
        uid.reshape(-1, 1), iid.reshape(-1, 1), ue, ie,
        W1[:_EMB], W1[_EMB:],
        b1.reshape(1, -1),
        W2, b2.reshape(1, -1),
        W3, b3.reshape(1, -1),
        Wout.T, bout.reshape(1, 1),
    )


# speedup vs baseline: 3.7670x; 1.5926x over previous
import jax
import jax.numpy as jnp
from jax import lax
from jax.experimental import pallas as pl
from jax.experimental.pallas import tpu as pltpu
from jax.experimental.pallas import tpu_sc as plsc

_BATCH = 16384
_EMB = 32
_NROWS = 1000000
_NC = 2
_NS = 16
_NW = _NC * _NS
_BPW = _BATCH // _NW
_CHUNK = 128
_NCHUNK = _BPW // _CHUNK
_LANES = 128

_TBLK = 16384
_TGRID = -(-_NROWS // (4 * _TBLK))
_Q = _TGRID * _TBLK


_TSAFE = (_NROWS - 3 * _Q) // _TBLK
_TAILC = _NROWS - (3 * _Q + _TSAFE * _TBLK)


def _tr_body(x0, x1, x2, x3, sliver, eye, out):
    x3v = jnp.where(pl.program_id(0) == _TSAFE, sliver[...], x3[...])
    xcat = jnp.concatenate([x0[...], x1[...], x2[...], x3v], axis=0)
    out[...] = jax.lax.dot_general(
        xcat, eye[...], (((0,), (0,)), ((), ())),
        preferred_element_type=jnp.float32)


@jax.jit
def _pack(tab_t):
    sliver = jnp.pad(tab_t[:, 3 * _Q + _TSAFE * _TBLK:],
                     ((0, 0), (0, _TBLK - _TAILC)))
    spec = lambda c: pl.BlockSpec((_EMB, _TBLK), lambda i, c=c: (0, i + _TGRID * c))
    c3spec = pl.BlockSpec((_EMB, _TBLK),
                          lambda i: (0, jnp.minimum(i, _TSAFE - 1) + 3 * _TGRID))
    return pl.pallas_call(
        _tr_body,
        grid=(_TGRID,),
        in_specs=[spec(0), spec(1), spec(2), c3spec,
                  pl.BlockSpec((_EMB, _TBLK), lambda i: (0, 0)),
                  pl.BlockSpec((_LANES, _LANES), lambda i: (0, 0))],
        out_specs=pl.BlockSpec((_TBLK, _LANES), lambda i: (i, 0)),
        out_shape=jax.ShapeDtypeStruct((_Q, _LANES), jnp.float32),
        compiler_params=pltpu.CompilerParams(
            dimension_semantics=("arbitrary",),
        ),
    )(tab_t, tab_t, tab_t, tab_t, sliver, jnp.eye(_LANES, dtype=jnp.float32))


def _gather_body(ids_hbm, tab_hbm, out_hbm, idx_v, gidx_v, buf, sem):
    wid = lax.axis_index("s") * _NC + lax.axis_index("c")
    base = wid * _BPW
    pltpu.sync_copy(ids_hbm.at[pl.ds(base, _BPW)], idx_v)
    for t in range(_BPW // 16):
        sl = pl.ds(t * 16, 16)
        r = idx_v[sl]
        c = ((r >= _Q).astype(jnp.int32) + (r >= 2 * _Q).astype(jnp.int32)
             + (r >= 3 * _Q).astype(jnp.int32))
        gidx_v[sl] = r - c * _Q
    handles = []
    for c in range(_NCHUNK):
        sl = pl.ds(c * _CHUNK, _CHUNK)
        handles.append(pltpu.async_copy(
            tab_hbm.at[gidx_v.at[sl]], buf.at[sl], sem))
    for h in handles:
        h.wait()
    pltpu.sync_copy(buf, out_hbm.at[pl.ds(base, _BPW)])


@jax.jit
def _gather(ids, tab2):
    mesh = plsc.VectorSubcoreMesh(core_axis_name="c", subcore_axis_name="s")
    k = pl.kernel(
        _gather_body,
        mesh=mesh,
        compiler_params=pltpu.CompilerParams(needs_layout_passes=False),
        out_type=jax.ShapeDtypeStruct((_BATCH, _LANES), jnp.float32),
        scratch_types=[
            pltpu.VMEM((_BPW,), jnp.int32),
            pltpu.VMEM((_BPW,), jnp.int32),
            pltpu.VMEM((_BPW, _LANES), jnp.float32),
            pltpu.SemaphoreType.DMA,
        ],
    )
    return k(ids, tab2)


_BLK = 2048


def _quarter(ids):
    return ((ids >= _Q).astype(jnp.int32) + (ids >= 2 * _Q).astype(jnp.int32)
            + (ids >= 3 * _Q).astype(jnp.int32))


def _select4(c, rows):
    return jnp.where(
        c < 2,
        jnp.where(c == 0, rows[:, 0 * _EMB:1 * _EMB], rows[:, 1 * _EMB:2 * _EMB]),
        jnp.where(c == 2, rows[:, 2 * _EMB:3 * _EMB], rows[:, 3 * _EMB:4 * _EMB]),
    )


def _mlp_body(uid_r, iid_r, ue, ie, w1u, w1i, b1, w2, b2, w3, b3, wo, bo, out):
    xu = _select4(_quarter(uid_r[...]), ue[...])
    xi = _select4(_quarter(iid_r[...]), ie[...])
    h = jnp.dot(xu, w1u[...], preferred_element_type=jnp.float32)
    h = h + jnp.dot(xi, w1i[...], preferred_element_type=jnp.float32)
    h = jnp.maximum(h + b1[...], 0.0)
    h = jnp.maximum(jnp.dot(h, w2[...], preferred_element_type=jnp.float32) + b2[...], 0.0)
    h = jnp.maximum(jnp.dot(h, w3[...], preferred_element_type=jnp.float32) + b3[...], 0.0)
    z = jnp.sum(h * wo[...], axis=1) + bo[0, 0]
    out[...] = 1.0 / (1.0 + jnp.exp(-z))


@jax.jit
def _mlp(uid2, iid2, ue, ie, w1u, w1i, b1, w2, b2, w3, b3, wo, bo):
    grid = (_BATCH // _BLK,)
    full = lambda shape: pl.BlockSpec(shape, lambda i: (0,) * len(shape))
    return pl.pallas_call(
        _mlp_body,
        grid=grid,
        in_specs=[
            pl.BlockSpec((_BLK, 1), lambda i: (i, 0)),
            pl.BlockSpec((_BLK, 1), lambda i: (i, 0)),
            pl.BlockSpec((_BLK, _LANES), lambda i: (i, 0)),
            pl.BlockSpec((_BLK, _LANES), lambda i: (i, 0)),
            full(w1u.shape),
            full(w1i.shape),
            full(b1.shape),
            full(w2.shape),
            full(b2.shape),
            full(w3.shape),
            full(b3.shape),
            full(wo.shape),
            full(bo.shape),
        ],
        out_specs=pl.BlockSpec((_BLK,), lambda i: (i,)),
        out_shape=jax.ShapeDtypeStruct((_BATCH,), jnp.float32),
        compiler_params=pltpu.CompilerParams(
            dimension_semantics=("arbitrary",),
        ),
    )(uid2, iid2, ue, ie, w1u, w1i, b1, w2, b2, w3, b3, wo, bo)


def kernel(user_ids, item_ids, user_table, item_table, W1, b1, W2, b2, W3, b3, Wout, bout):
    uid = user_ids.astype(jnp.int32)
    iid = item_ids.astype(jnp.int32)
    ut2 = _pack(user_table.T)
    it2 = _pack(item_table.T)
    ue = _gather(uid, ut2)
    ie = _gather(iid, it2)
    return _mlp(
        uid.reshape(-1, 1), iid.reshape(-1, 1), ue, ie,
        W1[:_EMB], W1[_EMB:],
        b1.reshape(1, -1),
        W2, b2.reshape(1, -1),
        W3, b3.reshape(1, -1),
        Wout.T, bout.reshape(1, 1),
    )
